# baseline (speedup 1.0000x reference)
import functools

import jax
import jax.numpy as jnp
from jax import lax
from jax.experimental import pallas as pl
from jax.experimental.pallas import tpu as pltpu
from jax.experimental.pallas import tpu_sc as plsc

B, H = 4096, 200
D = 128
N = B * H
NC, NS, L = 2, 16, 16
NW = NC * NS
RPW = N // NW
CH = 40
CPW = RPW // CH
NBUF = 4
OBUF = 2

HALF_PI = 1.5707963267948966
PI = 3.141592653589793
A0 = 0.9998383860193922
A1 = -0.326983305517636
A2 = 0.15936586312036266
A3 = -0.047260694565070184
SIGN_MASK = -2147483648


@functools.partial(
    pl.kernel,
    out_type=jax.ShapeDtypeStruct((N, 2 * D), jnp.float32),
    mesh=plsc.VectorSubcoreMesh(core_axis_name="c", subcore_axis_name="s"),
    scratch_types=[
        pltpu.VMEM((RPW,), jnp.int32),
        pltpu.VMEM((NBUF, CH, D), jnp.float32),
        pltpu.VMEM((NBUF, CH, D), jnp.float32),
        pltpu.VMEM((OBUF, CH, 2 * D), jnp.float32),
        pltpu.SemaphoreType.DMA,
        pltpu.SemaphoreType.DMA,
        pltpu.SemaphoreType.DMA,
        pltpu.SemaphoreType.DMA,
        pltpu.SemaphoreType.DMA,
        pltpu.SemaphoreType.DMA,
    ],
)
def _qemb(idx_hbm, real_hbm, imag_hbm, out_hbm, idx_all, re_v, im_v, ob,
          sg0, sg1, sg2, sg3, sw0, sw1):
    sem_g = (sg0, sg1, sg2, sg3)
    sem_w = (sw0, sw1)
    wid = lax.axis_index("s") * NC + lax.axis_index("c")
    wbase = wid * RPW

    pltpu.sync_copy(idx_hbm.at[wid], idx_all)

    def fire_gather(ci, b):
        ix = idx_all.at[pl.ds(ci * CH, CH)]
        pltpu.async_copy(real_hbm.at[ix], re_v.at[b], sem_g[b])
        pltpu.async_copy(imag_hbm.at[ix], im_v.at[b], sem_g[b])

    def wait_gather(ci, b):
        ix = idx_all.at[pl.ds(ci * CH, CH)]
        pltpu.make_async_copy(real_hbm.at[ix], re_v.at[b], sem_g[b]).wait()
        pltpu.make_async_copy(imag_hbm.at[ix], im_v.at[b], sem_g[b]).wait()

    def fire_write(ci, o):
        base = wbase + ci * CH
        pltpu.async_copy(ob.at[o], out_hbm.at[pl.ds(base, CH)], sem_w[o])

    def wait_write(ci, o):
        base = wbase + ci * CH
        pltpu.make_async_copy(
            ob.at[o], out_hbm.at[pl.ds(base, CH)], sem_w[o]).wait()

    def compute(b, o):
        def row_body(row, c2):
            for l in range(D // L):
                sl = pl.ds(l * L, L)
                r = re_v[b, row, sl]
                i = im_v[b, row, sl]
                x = r * r + i * i
                xi = lax.bitcast_convert_type(x, jnp.int32)
                y = lax.bitcast_convert_type(
                    jnp.int32(0x5F3759DF) - (xi >> 1), jnp.float32)
                y = y * (1.5 - (0.5 * x) * (y * y))
                ax = jnp.abs(r)
                ay = jnp.abs(i)
                mx = jnp.maximum(ax, ay)
                mn = jnp.minimum(ax, ay)
                nz = mx > 0.0
                mag = jnp.where(nz, x * y, 0.0)
                den = jnp.where(nz, mx, 1.0)
                t = mn / den
                u = t * t
                p = A3
                p = p * u + A2
                p = p * u + A1
                p = p * u + A0
                ph = p * t
                ph = jnp.where(ay > ax, HALF_PI - ph, ph)
                ph = jnp.where(r < 0.0, PI - ph, ph)
                ph = lax.bitcast_convert_type(
                    lax.bitcast_convert_type(ph, jnp.int32)
                    ^ (lax.bitcast_convert_type(i, jnp.int32) & SIGN_MASK),
                    jnp.float32)
                ob[o, row, sl] = mag
                ob[o, row, pl.ds(D + l * L, L)] = ph
            return c2

        lax.fori_loop(0, CH, row_body, 0, unroll=2)

    def substep(ci, b, o, do_wait_write, do_fire):
        wait_gather(ci, b)
        if do_fire:
            fire_gather(ci + 3, (b + 3) % NBUF)
        if do_wait_write:
            wait_write(ci - 2, o)
        compute(b, o)
        fire_write(ci, o)

    fire_gather(0, 0)
    fire_gather(1, 1)
    fire_gather(2, 2)

    substep(0, 0, 0, False, True)
    substep(1, 1, 1, False, True)
    substep(2, 2, 0, True, True)
    substep(3, 3, 1, True, True)

    def turn(k, carry):
        ci = k * NBUF
        for b in range(NBUF):
            substep(ci + b, b, b % OBUF, True, True)
        return carry

    lax.fori_loop(1, CPW // NBUF - 1, turn, 0, unroll=False)

    substep(CPW - 4, 0, 0, True, True)
    substep(CPW - 3, 1, 1, True, False)
    substep(CPW - 2, 2, 0, True, False)
    substep(CPW - 1, 3, 1, True, False)

    wait_write(CPW - 2, 0)
    wait_write(CPW - 1, 1)


def kernel(inputs, real_table, imag_table):
    idx = inputs.reshape(NW, RPW).astype(jnp.int32)
    out = _qemb(idx, real_table, imag_table)
    return out.reshape(B, H, 2 * D)


# speedup vs baseline: 1.6743x; 1.6743x over previous
import functools

import jax
import jax.numpy as jnp
from jax import lax
from jax.experimental import pallas as pl
from jax.experimental.pallas import tpu as pltpu
from jax.experimental.pallas import tpu_sc as plsc

B, H = 4096, 200
D = 128
N = B * H
NC, NS, L = 2, 16, 16
NW = NC * NS
RPW = N // NW
CH = 128
NCHUNK = RPW // CH

HALF_PI = 1.5707963267948966
PI = 3.141592653589793
A0 = 0.9998383860193922
A1 = -0.326983305517636
A2 = 0.15936586312036266
A3 = -0.047260694565070184
SIGN_MASK = -2147483648


@functools.partial(
    pl.kernel,
    out_type=jax.ShapeDtypeStruct((N, 2, D), jnp.float32),
    mesh=plsc.VectorSubcoreMesh(core_axis_name="c", subcore_axis_name="s"),
    scratch_types=[
        pltpu.VMEM((CH,), jnp.int32),
        pltpu.VMEM((CH, D), jnp.float32),
        pltpu.VMEM((CH, D), jnp.float32),
        pltpu.SemaphoreType.DMA,
        pltpu.SemaphoreType.DMA,
    ],
)
def _qemb(idx_hbm, real_hbm, imag_hbm, out_hbm, idx_v, re_v, im_v, sem_r, sem_i):
    wid = lax.axis_index("s") * NC + lax.axis_index("c")
    wbase = wid * RPW

    def chunk_body(ci, carry):
        base = wbase + ci * CH
        pltpu.sync_copy(idx_hbm.at[pl.ds(base, CH)], idx_v)
        cp_r = pltpu.async_copy(real_hbm.at[idx_v], re_v, sem_r)
        cp_i = pltpu.async_copy(imag_hbm.at[idx_v], im_v, sem_i)
        cp_r.wait()
        cp_i.wait()

        def row_body(row, c2):
            for l in range(D // L):
                sl = pl.ds(l * L, L)
                r = re_v[row, sl]
                i = im_v[row, sl]
                x = r * r + i * i
                xi = lax.bitcast_convert_type(x, jnp.int32)
                y = lax.bitcast_convert_type(
                    jnp.int32(0x5F3759DF) - (xi >> 1), jnp.float32)
                y = y * (1.5 - (0.5 * x) * (y * y))
                ax = jnp.abs(r)
                ay = jnp.abs(i)
                mx = jnp.maximum(ax, ay)
                mn = jnp.minimum(ax, ay)
                nz = mx > 0.0
                mag = jnp.where(nz, x * y, 0.0)
                den = jnp.where(nz, mx, 1.0)
                t = mn / den
                u = t * t
                p = A3
                p = p * u + A2
                p = p * u + A1
                p = p * u + A0
                ph = p * t
                ph = jnp.where(ay > ax, HALF_PI - ph, ph)
                ph = jnp.where(r < 0.0, PI - ph, ph)
                ph = lax.bitcast_convert_type(
                    lax.bitcast_convert_type(ph, jnp.int32)
                    ^ (lax.bitcast_convert_type(i, jnp.int32) & SIGN_MASK),
                    jnp.float32)
                re_v[row, sl] = mag
                im_v[row, sl] = ph
            return c2

        lax.fori_loop(0, CH, row_body, 0, unroll=False)
        pltpu.sync_copy(re_v, out_hbm.at[pl.ds(base, CH), 0])
        pltpu.sync_copy(im_v, out_hbm.at[pl.ds(base, CH), 1])
        return carry

    lax.fori_loop(0, NCHUNK, chunk_body, 0, unroll=False)


def kernel(inputs, real_table, imag_table):
    idx = inputs.reshape(N).astype(jnp.int32)
    out = _qemb(idx, real_table, imag_table)
    return out.reshape(B, H, 2 * D)
